# baseline (speedup 1.0000x reference)
import jax
import jax.numpy as jnp
from jax import lax
from jax.experimental import pallas as pl
from jax.experimental.pallas import tpu as pltpu
from jax.experimental.pallas import tpu_sc as plsc

N = 10000
E = 320000
D = 128
H = 128
OUT = 32
G = 64

try:
    _info = plsc.get_sparse_core_info()
    NC = _info.num_cores
    NS = _info.num_subcores
except ValueError:
    NC = 2
    NS = 16
NW = NC * NS

HD = D // NC
CH = 128
EPT = E // NS
NCHUNK = 160
PAD_E = NCHUNK * CH - EPT
NBUF = 5
ACC_ROWS = N + 8
ROWS_PER_TILE = 632
TAIL_ROWS = N - (NS - 1) * ROWS_PER_TILE

_mesh = plsc.VectorSubcoreMesh(
    core_axis_name="c", subcore_axis_name="s", num_cores=NC, num_subcores=NS
)


EPW = E // NW
NCHUNK_A = 79
PAD_A = NCHUNK_A * CH - EPW
DW = 16


def _deg_body(dst_r, zcol, onescol, pdeg, dstbuf, onesbuf, acc, sem0, sem1):
    c = lax.axis_index("c")
    s = lax.axis_index("s")
    w = c * NS + s

    @pl.when(s == 0)
    def _():
        pltpu.sync_copy(zcol, acc.at[pl.ds(0, N)])

    pltpu.sync_copy(dst_r.at[w], dstbuf)
    pltpu.sync_copy(onescol, onesbuf)
    plsc.subcore_barrier()

    sems = [sem0, sem1]

    def fire(g, b):
        pltpu.async_copy(onesbuf, acc.at[dstbuf.at[g]], sems[b], add=True)

    def drain(b):
        pltpu.make_async_copy(onesbuf, acc.at[dstbuf.at[0]], sems[b]).wait()

    fire(0, 0)
    fire(1, 1)

    def loop(gi, carry):
        for b in range(2):
            g = 2 + gi * 2 + b
            drain(b)
            fire(g, b)
        return carry

    lax.fori_loop(0, (NCHUNK_A - 2) // 2, loop, 0)
    for r in range((NCHUNK_A - 2) % 2):
        drain(r)
        fire(NCHUNK_A - 1, r)
    drain(0)
    drain(1)

    plsc.subcore_barrier()

    @pl.when(s == 0)
    def _():
        pltpu.sync_copy(acc.at[pl.ds(0, N)], pdeg.at[c])


_deg_call = pl.kernel(
    _deg_body,
    out_type=jax.ShapeDtypeStruct((NC, N, DW), jnp.float32),
    mesh=_mesh,
    scratch_types=[
        pltpu.VMEM((NCHUNK_A, CH), jnp.int32),
        pltpu.VMEM((CH, DW), jnp.float32),
        pltpu.VMEM_SHARED((ACC_ROWS, DW), jnp.float32),
        pltpu.SemaphoreType.DMA,
        pltpu.SemaphoreType.DMA,
    ],
    compiler_params=pltpu.CompilerParams(use_tc_tiling_on_sc=False),
)


def _agg_body(uh, src_r, dst_r, zrows, parts, srcbuf, dstbuf, rows, acc,
              g0, g1, g2, g3, g4):
    c = lax.axis_index("c")
    s = lax.axis_index("s")

    tab = uh.at[c]

    @pl.when(s < NS - 1)
    def _():
        pltpu.sync_copy(zrows,
                        acc.at[pl.ds(s * ROWS_PER_TILE, ROWS_PER_TILE)])

    @pl.when(s == NS - 1)
    def _():
        pltpu.sync_copy(
            zrows.at[pl.ds(0, TAIL_ROWS)],
            acc.at[pl.ds((NS - 1) * ROWS_PER_TILE, TAIL_ROWS)],
        )

    pltpu.sync_copy(src_r.at[s], srcbuf)
    pltpu.sync_copy(dst_r.at[s], dstbuf)
    plsc.subcore_barrier()

    gsems = [g0, g1, g2, g3, g4]

    def fire(g, b):
        pltpu.async_copy(tab.at[srcbuf.at[g]], rows.at[b], gsems[b])

    def wait_gather(b):
        pltpu.make_async_copy(tab.at[pl.ds(0, CH)], rows.at[b],
                              gsems[b]).wait()

    def sync_scatter(g, b):
        pltpu.sync_copy(rows.at[b], acc.at[dstbuf.at[g]], add=True)

    for b in range(NBUF):
        fire(b, b)

    def loop(gi, carry):
        for b in range(NBUF):
            g = gi * NBUF + b
            wait_gather(b)
            sync_scatter(g, b)
            fire(g + NBUF, b)
        return carry

    lax.fori_loop(0, NCHUNK // NBUF - 1, loop, 0)
    for b in range(NBUF):
        g = NCHUNK - NBUF + b
        wait_gather(b)
        sync_scatter(g, b)

    plsc.subcore_barrier()
    tail = N - (NS - 1) * ROWS_PER_TILE

    @pl.when(s < NS - 1)
    def _():
        pltpu.sync_copy(
            acc.at[pl.ds(s * ROWS_PER_TILE, ROWS_PER_TILE)],
            parts.at[c, pl.ds(s * ROWS_PER_TILE, ROWS_PER_TILE), :],
        )

    @pl.when(s == NS - 1)
    def _():
        pltpu.sync_copy(
            acc.at[pl.ds((NS - 1) * ROWS_PER_TILE, tail)],
            parts.at[c, pl.ds((NS - 1) * ROWS_PER_TILE, tail), :],
        )


_agg_call = pl.kernel(
    _agg_body,
    out_type=jax.ShapeDtypeStruct((NC, N, HD), jnp.float32),
    mesh=_mesh,
    scratch_types=[
        pltpu.VMEM((NCHUNK, CH), jnp.int32),
        pltpu.VMEM((NCHUNK, CH), jnp.int32),
        pltpu.VMEM((NBUF, CH, HD), jnp.float32),
        pltpu.VMEM_SHARED((ACC_ROWS, HD), jnp.float32),
    ] + [pltpu.SemaphoreType.DMA] * 5,
    compiler_params=pltpu.CompilerParams(use_tc_tiling_on_sc=False),
)


RB = 2000


def _b_body(pd_ref, x_ref, w_ref, u_ref, dinv_ref):
    deg = pd_ref[0][:, 0:1] + pd_ref[1][:, 0:1] + 1.0
    dinv = lax.rsqrt(deg)
    res = (
        jnp.dot(x_ref[...], w_ref[...], preferred_element_type=jnp.float32)
        * dinv
    )
    u_ref[0] = res[:, :HD]
    u_ref[1] = res[:, HD:]
    dinv_ref[...] = dinv


def _call_b(pdeg, x, W1):
    grid = (N // RB,)
    return pl.pallas_call(
        _b_body,
        grid=grid,
        in_specs=[
            pl.BlockSpec((NC, RB, DW), lambda i: (0, i, 0)),
            pl.BlockSpec((RB, D), lambda i: (i, 0)),
            pl.BlockSpec((D, H), lambda i: (0, 0)),
        ],
        out_specs=[
            pl.BlockSpec((NC, RB, HD), lambda i: (0, i, 0)),
            pl.BlockSpec((RB, 1), lambda i: (i, 0)),
        ],
        out_shape=[
            jax.ShapeDtypeStruct((NC, N, HD), jnp.float32),
            jax.ShapeDtypeStruct((N, 1), jnp.float32),
        ],
    )(pdeg, x, W1)


def _d_body(p_ref, u_ref, dinv_ref, b1_ref, w2_ref, u2_ref):
    pres = jnp.concatenate(
        [p_ref[0] + u_ref[0], p_ref[1] + u_ref[1]], axis=1
    )
    h1 = jnp.maximum(dinv_ref[...] * pres + b1_ref[...], 0.0)
    res = (
        jnp.dot(h1, w2_ref[...], preferred_element_type=jnp.float32)
        * dinv_ref[...]
    )
    u2_ref[0] = res[:, :HD]
    u2_ref[1] = res[:, HD:]


def _call_d(parts, u1, dinv, b1, W2):
    grid = (N // RB,)
    return pl.pallas_call(
        _d_body,
        grid=grid,
        in_specs=[
            pl.BlockSpec((NC, RB, HD), lambda i: (0, i, 0)),
            pl.BlockSpec((NC, RB, HD), lambda i: (0, i, 0)),
            pl.BlockSpec((RB, 1), lambda i: (i, 0)),
            pl.BlockSpec((1, H), lambda i: (0, 0)),
            pl.BlockSpec((H, H), lambda i: (0, 0)),
        ],
        out_specs=pl.BlockSpec((NC, RB, HD), lambda i: (0, i, 0)),
        out_shape=jax.ShapeDtypeStruct((NC, N, HD), jnp.float32),
    )(parts, u1, dinv, b1, W2)


def _e_body(p_ref, u2_ref, dinv_ref, b2_ref, batch_ref, l1w_ref, l1b_ref,
            l2w_ref, l2b_ref, out_ref, psum, cnt):
    i = pl.program_id(0)
    pres = jnp.concatenate(
        [p_ref[0] + u2_ref[0], p_ref[1] + u2_ref[1]], axis=1
    )
    h2 = jnp.maximum(dinv_ref[...] * pres + b2_ref[...], 0.0)
    bt = batch_ref[0]
    oh = (lax.broadcasted_iota(jnp.int32, (G, RB), 0) == bt).astype(
        jnp.float32
    )
    ps = jnp.dot(oh, h2, preferred_element_type=jnp.float32)
    cn = jnp.sum(oh, axis=1, keepdims=True)

    @pl.when(i == 0)
    def _():
        psum[...] = ps
        cnt[...] = cn

    @pl.when(i > 0)
    def _():
        psum[...] += ps
        cnt[...] += cn

    @pl.when(i == (N // RB) - 1)
    def _():
        pooled = psum[...] / jnp.maximum(cnt[...], 1.0)
        xf = (
            jnp.dot(pooled, l1w_ref[...], preferred_element_type=jnp.float32)
            + l1b_ref[...]
        )
        out_ref[...] = (
            jnp.dot(
                jnp.maximum(xf, 0.0),
                l2w_ref[...],
                preferred_element_type=jnp.float32,
            )
            + l2b_ref[...]
        )


def _call_e(parts, u2, dinv, b2, batch3, lin1_W, lin1_b, lin2_W, lin2_b):
    grid = (N // RB,)
    return pl.pallas_call(
        _e_body,
        grid=grid,
        in_specs=[
            pl.BlockSpec((NC, RB, HD), lambda i: (0, i, 0)),
            pl.BlockSpec((NC, RB, HD), lambda i: (0, i, 0)),
            pl.BlockSpec((RB, 1), lambda i: (i, 0)),
            pl.BlockSpec((1, H), lambda i: (0, 0)),
            pl.BlockSpec((1, 1, RB), lambda i: (i, 0, 0)),
            pl.BlockSpec((H, D), lambda i: (0, 0)),
            pl.BlockSpec((1, D), lambda i: (0, 0)),
            pl.BlockSpec((D, OUT), lambda i: (0, 0)),
            pl.BlockSpec((1, OUT), lambda i: (0, 0)),
        ],
        out_specs=pl.BlockSpec((G, OUT), lambda i: (0, 0)),
        out_shape=jax.ShapeDtypeStruct((G, OUT), jnp.float32),
        scratch_shapes=[
            pltpu.VMEM((G, H), jnp.float32),
            pltpu.VMEM((G, 1), jnp.float32),
        ],
    )(parts, u2, dinv, b2, batch3, lin1_W, lin1_b, lin2_W, lin2_b)


def kernel(x, edge_index, batch, W1, b1, W2, b2, lin1_W, lin1_b, lin2_W,
           lin2_b):
    src = edge_index[0].astype(jnp.int32)
    dst = edge_index[1].astype(jnp.int32)
    src_r = jnp.concatenate(
        [src.reshape(NS, EPT), jnp.zeros((NS, PAD_E), jnp.int32)], axis=1
    ).reshape(NS, NCHUNK, CH)
    dst_r = jnp.concatenate(
        [dst.reshape(NS, EPT), jnp.full((NS, PAD_E), N, jnp.int32)], axis=1
    ).reshape(NS, NCHUNK, CH)
    dst_a = jnp.concatenate(
        [dst.reshape(NW, EPW), jnp.full((NW, PAD_A), N, jnp.int32)], axis=1
    ).reshape(NW, NCHUNK_A, CH)

    zcol = jnp.zeros((N, DW), jnp.float32)
    onescol = jnp.ones((CH, DW), jnp.float32)
    zrows = jnp.zeros((ROWS_PER_TILE, HD), jnp.float32)

    pdeg = _deg_call(dst_a, zcol, onescol)
    u1, dinv = _call_b(pdeg, x, W1)
    p1 = _agg_call(u1, src_r, dst_r, zrows)
    u2 = _call_d(p1, u1, dinv, b1.reshape(1, H), W2)
    p2 = _agg_call(u2, src_r, dst_r, zrows)
    out = _call_e(
        p2, u2, dinv, b2.reshape(1, H),
        batch.reshape(N // RB, 1, RB).astype(jnp.int32),
        lin1_W, lin1_b.reshape(1, D), lin2_W, lin2_b.reshape(1, OUT),
    )
    return out


# speedup vs baseline: 1.0008x; 1.0008x over previous
import jax
import jax.numpy as jnp
from jax import lax
from jax.experimental import pallas as pl
from jax.experimental.pallas import tpu as pltpu
from jax.experimental.pallas import tpu_sc as plsc

N = 10000
E = 320000
D = 128
H = 128
OUT = 32
G = 64

try:
    _info = plsc.get_sparse_core_info()
    NC = _info.num_cores
    NS = _info.num_subcores
except ValueError:
    NC = 2
    NS = 16
NW = NC * NS

HD = D // NC
CH = 128
EPT = E // NS
NCHUNK = 160
PAD_E = NCHUNK * CH - EPT
NBUF = 5
ACC_ROWS = N + 8
ROWS_PER_TILE = 640
TAIL_ROWS = N - (NS - 1) * ROWS_PER_TILE

_mesh = plsc.VectorSubcoreMesh(
    core_axis_name="c", subcore_axis_name="s", num_cores=NC, num_subcores=NS
)


EPW = E // NW
NCHUNK_A = 79
PAD_A = NCHUNK_A * CH - EPW
DW = 16


def _deg_body(dst_r, zcol, onescol, pdeg, dstbuf, onesbuf, acc, sem0, sem1):
    c = lax.axis_index("c")
    s = lax.axis_index("s")
    w = c * NS + s

    @pl.when(s == 0)
    def _():
        pltpu.sync_copy(zcol, acc.at[pl.ds(0, N)])

    pltpu.sync_copy(dst_r.at[w], dstbuf)
    pltpu.sync_copy(onescol, onesbuf)
    plsc.subcore_barrier()

    sems = [sem0, sem1]

    def fire(g, b):
        pltpu.async_copy(onesbuf, acc.at[dstbuf.at[g]], sems[b], add=True)

    def drain(b):
        pltpu.make_async_copy(onesbuf, acc.at[dstbuf.at[0]], sems[b]).wait()

    fire(0, 0)
    fire(1, 1)

    def loop(gi, carry):
        for b in range(2):
            g = 2 + gi * 2 + b
            drain(b)
            fire(g, b)
        return carry

    lax.fori_loop(0, (NCHUNK_A - 2) // 2, loop, 0)
    for r in range((NCHUNK_A - 2) % 2):
        drain(r)
        fire(NCHUNK_A - 1, r)
    drain(0)
    drain(1)

    plsc.subcore_barrier()

    @pl.when(s == 0)
    def _():
        pltpu.sync_copy(acc.at[pl.ds(0, N)], pdeg.at[c])


_deg_call = pl.kernel(
    _deg_body,
    out_type=jax.ShapeDtypeStruct((NC, N, DW), jnp.float32),
    mesh=_mesh,
    scratch_types=[
        pltpu.VMEM((NCHUNK_A, CH), jnp.int32),
        pltpu.VMEM((CH, DW), jnp.float32),
        pltpu.VMEM_SHARED((ACC_ROWS, DW), jnp.float32),
        pltpu.SemaphoreType.DMA,
        pltpu.SemaphoreType.DMA,
    ],
    compiler_params=pltpu.CompilerParams(use_tc_tiling_on_sc=False),
)


def _agg_body(uh, src_r, dst_r, zrows, parts, srcbuf, dstbuf, rows, acc,
              g0, g1, g2, g3, g4):
    c = lax.axis_index("c")
    s = lax.axis_index("s")

    tab = uh.at[c]

    @pl.when(s < NS - 1)
    def _():
        pltpu.sync_copy(zrows,
                        acc.at[pl.ds(s * ROWS_PER_TILE, ROWS_PER_TILE)])

    @pl.when(s == NS - 1)
    def _():
        pltpu.sync_copy(
            zrows.at[pl.ds(0, TAIL_ROWS)],
            acc.at[pl.ds((NS - 1) * ROWS_PER_TILE, TAIL_ROWS)],
        )

    pltpu.sync_copy(src_r.at[s], srcbuf)
    pltpu.sync_copy(dst_r.at[s], dstbuf)
    plsc.subcore_barrier()

    gsems = [g0, g1, g2, g3, g4]

    def fire(g, b):
        pltpu.async_copy(tab.at[srcbuf.at[g]], rows.at[b], gsems[b])

    def wait_gather(b):
        pltpu.make_async_copy(tab.at[pl.ds(0, CH)], rows.at[b],
                              gsems[b]).wait()

    def sync_scatter(g, b):
        pltpu.sync_copy(rows.at[b], acc.at[dstbuf.at[g]], add=True)

    for b in range(NBUF):
        fire(b, b)

    def loop(gi, carry):
        for b in range(NBUF):
            g = gi * NBUF + b
            wait_gather(b)
            sync_scatter(g, b)
            fire(g + NBUF, b)
        return carry

    lax.fori_loop(0, NCHUNK // NBUF - 1, loop, 0)
    for b in range(NBUF):
        g = NCHUNK - NBUF + b
        wait_gather(b)
        sync_scatter(g, b)

    plsc.subcore_barrier()
    tail = N - (NS - 1) * ROWS_PER_TILE

    @pl.when(s < NS - 1)
    def _():
        pltpu.sync_copy(
            acc.at[pl.ds(s * ROWS_PER_TILE, ROWS_PER_TILE)],
            parts.at[c, pl.ds(s * ROWS_PER_TILE, ROWS_PER_TILE), :],
        )

    @pl.when(s == NS - 1)
    def _():
        pltpu.sync_copy(
            acc.at[pl.ds((NS - 1) * ROWS_PER_TILE, tail)],
            parts.at[c, pl.ds((NS - 1) * ROWS_PER_TILE, tail), :],
        )


_agg_call = pl.kernel(
    _agg_body,
    out_type=jax.ShapeDtypeStruct((NC, N, HD), jnp.float32),
    mesh=_mesh,
    scratch_types=[
        pltpu.VMEM((NCHUNK, CH), jnp.int32),
        pltpu.VMEM((NCHUNK, CH), jnp.int32),
        pltpu.VMEM((NBUF, CH, HD), jnp.float32),
        pltpu.VMEM_SHARED((ACC_ROWS, HD), jnp.float32),
    ] + [pltpu.SemaphoreType.DMA] * 5,
    compiler_params=pltpu.CompilerParams(use_tc_tiling_on_sc=False),
)


RB = 2000


def _b_body(pd_ref, x_ref, w_ref, u_ref, dinv_ref):
    deg = pd_ref[0][:, 0:1] + pd_ref[1][:, 0:1] + 1.0
    dinv = lax.rsqrt(deg)
    res = (
        jnp.dot(x_ref[...], w_ref[...], preferred_element_type=jnp.float32)
        * dinv
    )
    u_ref[0] = res[:, :HD]
    u_ref[1] = res[:, HD:]
    dinv_ref[...] = dinv


def _call_b(pdeg, x, W1):
    grid = (N // RB,)
    return pl.pallas_call(
        _b_body,
        grid=grid,
        in_specs=[
            pl.BlockSpec((NC, RB, DW), lambda i: (0, i, 0)),
            pl.BlockSpec((RB, D), lambda i: (i, 0)),
            pl.BlockSpec((D, H), lambda i: (0, 0)),
        ],
        out_specs=[
            pl.BlockSpec((NC, RB, HD), lambda i: (0, i, 0)),
            pl.BlockSpec((RB, 1), lambda i: (i, 0)),
        ],
        out_shape=[
            jax.ShapeDtypeStruct((NC, N, HD), jnp.float32),
            jax.ShapeDtypeStruct((N, 1), jnp.float32),
        ],
    )(pdeg, x, W1)


def _d_body(p_ref, u_ref, dinv_ref, b1_ref, w2_ref, u2_ref):
    pres = jnp.concatenate(
        [p_ref[0] + u_ref[0], p_ref[1] + u_ref[1]], axis=1
    )
    h1 = jnp.maximum(dinv_ref[...] * pres + b1_ref[...], 0.0)
    res = (
        jnp.dot(h1, w2_ref[...], preferred_element_type=jnp.float32)
        * dinv_ref[...]
    )
    u2_ref[0] = res[:, :HD]
    u2_ref[1] = res[:, HD:]


def _call_d(parts, u1, dinv, b1, W2):
    grid = (N // RB,)
    return pl.pallas_call(
        _d_body,
        grid=grid,
        in_specs=[
            pl.BlockSpec((NC, RB, HD), lambda i: (0, i, 0)),
            pl.BlockSpec((NC, RB, HD), lambda i: (0, i, 0)),
            pl.BlockSpec((RB, 1), lambda i: (i, 0)),
            pl.BlockSpec((1, H), lambda i: (0, 0)),
            pl.BlockSpec((H, H), lambda i: (0, 0)),
        ],
        out_specs=pl.BlockSpec((NC, RB, HD), lambda i: (0, i, 0)),
        out_shape=jax.ShapeDtypeStruct((NC, N, HD), jnp.float32),
    )(parts, u1, dinv, b1, W2)


def _e_body(p_ref, u2_ref, dinv_ref, b2_ref, batch_ref, l1w_ref, l1b_ref,
            l2w_ref, l2b_ref, out_ref, psum, cnt):
    i = pl.program_id(0)
    pres = jnp.concatenate(
        [p_ref[0] + u2_ref[0], p_ref[1] + u2_ref[1]], axis=1
    )
    h2 = jnp.maximum(dinv_ref[...] * pres + b2_ref[...], 0.0)
    bt = batch_ref[0]
    oh = (lax.broadcasted_iota(jnp.int32, (G, RB), 0) == bt).astype(
        jnp.float32
    )
    ps = jnp.dot(oh, h2, preferred_element_type=jnp.float32)
    cn = jnp.sum(oh, axis=1, keepdims=True)

    @pl.when(i == 0)
    def _():
        psum[...] = ps
        cnt[...] = cn

    @pl.when(i > 0)
    def _():
        psum[...] += ps
        cnt[...] += cn

    @pl.when(i == (N // RB) - 1)
    def _():
        pooled = psum[...] / jnp.maximum(cnt[...], 1.0)
        xf = (
            jnp.dot(pooled, l1w_ref[...], preferred_element_type=jnp.float32)
            + l1b_ref[...]
        )
        out_ref[...] = (
            jnp.dot(
                jnp.maximum(xf, 0.0),
                l2w_ref[...],
                preferred_element_type=jnp.float32,
            )
            + l2b_ref[...]
        )


def _call_e(parts, u2, dinv, b2, batch3, lin1_W, lin1_b, lin2_W, lin2_b):
    grid = (N // RB,)
    return pl.pallas_call(
        _e_body,
        grid=grid,
        in_specs=[
            pl.BlockSpec((NC, RB, HD), lambda i: (0, i, 0)),
            pl.BlockSpec((NC, RB, HD), lambda i: (0, i, 0)),
            pl.BlockSpec((RB, 1), lambda i: (i, 0)),
            pl.BlockSpec((1, H), lambda i: (0, 0)),
            pl.BlockSpec((1, 1, RB), lambda i: (i, 0, 0)),
            pl.BlockSpec((H, D), lambda i: (0, 0)),
            pl.BlockSpec((1, D), lambda i: (0, 0)),
            pl.BlockSpec((D, OUT), lambda i: (0, 0)),
            pl.BlockSpec((1, OUT), lambda i: (0, 0)),
        ],
        out_specs=pl.BlockSpec((G, OUT), lambda i: (0, 0)),
        out_shape=jax.ShapeDtypeStruct((G, OUT), jnp.float32),
        scratch_shapes=[
            pltpu.VMEM((G, H), jnp.float32),
            pltpu.VMEM((G, 1), jnp.float32),
        ],
    )(parts, u2, dinv, b2, batch3, lin1_W, lin1_b, lin2_W, lin2_b)


def kernel(x, edge_index, batch, W1, b1, W2, b2, lin1_W, lin1_b, lin2_W,
           lin2_b):
    src = edge_index[0].astype(jnp.int32)
    dst = edge_index[1].astype(jnp.int32)
    src_r = jnp.concatenate(
        [src.reshape(NS, EPT), jnp.zeros((NS, PAD_E), jnp.int32)], axis=1
    ).reshape(NS, NCHUNK, CH)
    dst_r = jnp.concatenate(
        [dst.reshape(NS, EPT), jnp.full((NS, PAD_E), N, jnp.int32)], axis=1
    ).reshape(NS, NCHUNK, CH)
    dst_a = jnp.concatenate(
        [dst.reshape(NW, EPW), jnp.full((NW, PAD_A), N, jnp.int32)], axis=1
    ).reshape(NW, NCHUNK_A, CH)

    zcol = jnp.zeros((N, DW), jnp.float32)
    onescol = jnp.ones((CH, DW), jnp.float32)
    zrows = jnp.zeros((ROWS_PER_TILE, HD), jnp.float32)

    pdeg = _deg_call(dst_a, zcol, onescol)
    u1, dinv = _call_b(pdeg, x, W1)
    p1 = _agg_call(u1, src_r, dst_r, zrows)
    u2 = _call_d(p1, u1, dinv, b1.reshape(1, H), W2)
    p2 = _agg_call(u2, src_r, dst_r, zrows)
    out = _call_e(
        p2, u2, dinv, b2.reshape(1, H),
        batch.reshape(N // RB, 1, RB).astype(jnp.int32),
        lin1_W, lin1_b.reshape(1, D), lin2_W, lin2_b.reshape(1, OUT),
    )
    return out


# speedup vs baseline: 1.9214x; 1.9198x over previous
import jax
import jax.numpy as jnp
from jax import lax
from jax.experimental import pallas as pl
from jax.experimental.pallas import tpu as pltpu
from jax.experimental.pallas import tpu_sc as plsc

N = 10000
E = 320000
D = 128
H = 128
OUT = 32
G = 64

try:
    _info = plsc.get_sparse_core_info()
    NC = _info.num_cores
    NS = _info.num_subcores
except ValueError:
    NC = 2
    NS = 16
NW = NC * NS

HD = D // NC
CH = 128
CHA = 80
EPT = E // NS
NCHUNK = 250
PAD_E = NCHUNK * CHA - EPT
NBUF = 5
ACC_ROWS = N + 8
ROWS_PER_TILE = 640
TAIL_ROWS = N - (NS - 1) * ROWS_PER_TILE

_mesh = plsc.VectorSubcoreMesh(
    core_axis_name="c", subcore_axis_name="s", num_cores=NC, num_subcores=NS
)


EPW = E // NW
NCHUNK_A = 79
PAD_A = NCHUNK_A * CH - EPW
DW = 16


def _deg_body(dst_r, zcol, onescol, pdeg, dstbuf, onesbuf, acc, sem0, sem1):
    c = lax.axis_index("c")
    s = lax.axis_index("s")
    w = c * NS + s

    @pl.when(s == 0)
    def _():
        pltpu.sync_copy(zcol, acc.at[pl.ds(0, N)])

    pltpu.sync_copy(dst_r.at[w], dstbuf)
    pltpu.sync_copy(onescol, onesbuf)
    plsc.subcore_barrier()

    sems = [sem0, sem1]

    def fire(g, b):
        pltpu.async_copy(onesbuf, acc.at[dstbuf.at[g]], sems[b], add=True)

    def drain(b):
        pltpu.make_async_copy(onesbuf, acc.at[dstbuf.at[0]], sems[b]).wait()

    fire(0, 0)
    fire(1, 1)

    def loop(gi, carry):
        for b in range(2):
            g = 2 + gi * 2 + b
            drain(b)
            fire(g, b)
        return carry

    lax.fori_loop(0, (NCHUNK_A - 2) // 2, loop, 0)
    for r in range((NCHUNK_A - 2) % 2):
        drain(r)
        fire(NCHUNK_A - 1, r)
    drain(0)
    drain(1)

    plsc.subcore_barrier()

    @pl.when(s == 0)
    def _():
        pltpu.sync_copy(acc.at[pl.ds(0, N)], pdeg.at[c])


_deg_call = pl.kernel(
    _deg_body,
    out_type=jax.ShapeDtypeStruct((NC, N, DW), jnp.float32),
    mesh=_mesh,
    scratch_types=[
        pltpu.VMEM((NCHUNK_A, CH), jnp.int32),
        pltpu.VMEM((CH, DW), jnp.float32),
        pltpu.VMEM_SHARED((ACC_ROWS, DW), jnp.float32),
        pltpu.SemaphoreType.DMA,
        pltpu.SemaphoreType.DMA,
    ],
    compiler_params=pltpu.CompilerParams(use_tc_tiling_on_sc=False),
)


def _agg_body(uh, src_r, dst_r, zrows, parts, srcbuf, dstbuf, rows, acc,
              g0, g1, g2, g3, g4):
    c = lax.axis_index("c")
    s = lax.axis_index("s")

    tab = uh.at[c]

    @pl.when(s < NS - 1)
    def _():
        pltpu.sync_copy(zrows,
                        acc.at[pl.ds(s * ROWS_PER_TILE, ROWS_PER_TILE)])

    @pl.when(s == NS - 1)
    def _():
        pltpu.sync_copy(
            zrows.at[pl.ds(0, TAIL_ROWS)],
            acc.at[pl.ds((NS - 1) * ROWS_PER_TILE, TAIL_ROWS)],
        )

    pltpu.sync_copy(src_r.at[s], srcbuf)
    pltpu.sync_copy(dst_r.at[s], dstbuf)
    plsc.subcore_barrier()

    gsems = [g0, g1, g2, g3, g4]

    def fire(g, b):
        pltpu.async_copy(tab.at[srcbuf.at[g]], rows.at[b], gsems[b])

    def wait_gather(b):
        pltpu.make_async_copy(tab.at[pl.ds(0, CHA)], rows.at[b],
                              gsems[b]).wait()

    def sync_scatter(g, b):
        pltpu.sync_copy(rows.at[b], acc.at[dstbuf.at[g]], add=True)

    for b in range(NBUF):
        fire(b, b)

    def loop(gi, carry):
        for b in range(NBUF):
            g = gi * NBUF + b
            wait_gather(b)
            sync_scatter(g, b)
            fire(g + NBUF, b)
        return carry

    lax.fori_loop(0, NCHUNK // NBUF - 1, loop, 0)
    for b in range(NBUF):
        g = NCHUNK - NBUF + b
        wait_gather(b)
        sync_scatter(g, b)

    plsc.subcore_barrier()
    tail = N - (NS - 1) * ROWS_PER_TILE

    @pl.when(s < NS - 1)
    def _():
        pltpu.sync_copy(
            acc.at[pl.ds(s * ROWS_PER_TILE, ROWS_PER_TILE)],
            parts.at[c, pl.ds(s * ROWS_PER_TILE, ROWS_PER_TILE), :],
        )

    @pl.when(s == NS - 1)
    def _():
        pltpu.sync_copy(
            acc.at[pl.ds((NS - 1) * ROWS_PER_TILE, tail)],
            parts.at[c, pl.ds((NS - 1) * ROWS_PER_TILE, tail), :],
        )


_agg_call = pl.kernel(
    _agg_body,
    out_type=jax.ShapeDtypeStruct((NC, N, HD), jnp.float32),
    mesh=_mesh,
    scratch_types=[
        pltpu.VMEM((NCHUNK, CHA), jnp.int32),
        pltpu.VMEM((NCHUNK, CHA), jnp.int32),
        pltpu.VMEM((NBUF, CHA, HD), jnp.float32),
        pltpu.VMEM_SHARED((ACC_ROWS, HD), jnp.float32),
    ] + [pltpu.SemaphoreType.DMA] * 5,
    compiler_params=pltpu.CompilerParams(use_tc_tiling_on_sc=False),
)


RB = 2000


def _b_body(pd_ref, x_ref, w_ref, u_ref, dinv_ref):
    deg = pd_ref[0][:, 0:1] + pd_ref[1][:, 0:1] + 1.0
    dinv = lax.rsqrt(deg)
    res = (
        jnp.dot(x_ref[...], w_ref[...], preferred_element_type=jnp.float32)
        * dinv
    )
    u_ref[0] = res[:, :HD]
    u_ref[1] = res[:, HD:]
    dinv_ref[...] = dinv


def _call_b(pdeg, x, W1):
    grid = (N // RB,)
    return pl.pallas_call(
        _b_body,
        grid=grid,
        in_specs=[
            pl.BlockSpec((NC, RB, DW), lambda i: (0, i, 0)),
            pl.BlockSpec((RB, D), lambda i: (i, 0)),
            pl.BlockSpec((D, H), lambda i: (0, 0)),
        ],
        out_specs=[
            pl.BlockSpec((NC, RB, HD), lambda i: (0, i, 0)),
            pl.BlockSpec((RB, 1), lambda i: (i, 0)),
        ],
        out_shape=[
            jax.ShapeDtypeStruct((NC, N, HD), jnp.float32),
            jax.ShapeDtypeStruct((N, 1), jnp.float32),
        ],
    )(pdeg, x, W1)


def _d_body(p_ref, u_ref, dinv_ref, b1_ref, w2_ref, u2_ref):
    pres = jnp.concatenate(
        [p_ref[0] + u_ref[0], p_ref[1] + u_ref[1]], axis=1
    )
    h1 = jnp.maximum(dinv_ref[...] * pres + b1_ref[...], 0.0)
    res = (
        jnp.dot(h1, w2_ref[...], preferred_element_type=jnp.float32)
        * dinv_ref[...]
    )
    u2_ref[0] = res[:, :HD]
    u2_ref[1] = res[:, HD:]


def _call_d(parts, u1, dinv, b1, W2):
    grid = (N // RB,)
    return pl.pallas_call(
        _d_body,
        grid=grid,
        in_specs=[
            pl.BlockSpec((NC, RB, HD), lambda i: (0, i, 0)),
            pl.BlockSpec((NC, RB, HD), lambda i: (0, i, 0)),
            pl.BlockSpec((RB, 1), lambda i: (i, 0)),
            pl.BlockSpec((1, H), lambda i: (0, 0)),
            pl.BlockSpec((H, H), lambda i: (0, 0)),
        ],
        out_specs=pl.BlockSpec((NC, RB, HD), lambda i: (0, i, 0)),
        out_shape=jax.ShapeDtypeStruct((NC, N, HD), jnp.float32),
    )(parts, u1, dinv, b1, W2)


def _e_body(p_ref, u2_ref, dinv_ref, b2_ref, batch_ref, l1w_ref, l1b_ref,
            l2w_ref, l2b_ref, out_ref, psum, cnt):
    i = pl.program_id(0)
    pres = jnp.concatenate(
        [p_ref[0] + u2_ref[0], p_ref[1] + u2_ref[1]], axis=1
    )
    h2 = jnp.maximum(dinv_ref[...] * pres + b2_ref[...], 0.0)
    bt = batch_ref[0]
    oh = (lax.broadcasted_iota(jnp.int32, (G, RB), 0) == bt).astype(
        jnp.float32
    )
    ps = jnp.dot(oh, h2, preferred_element_type=jnp.float32)
    cn = jnp.sum(oh, axis=1, keepdims=True)

    @pl.when(i == 0)
    def _():
        psum[...] = ps
        cnt[...] = cn

    @pl.when(i > 0)
    def _():
        psum[...] += ps
        cnt[...] += cn

    @pl.when(i == (N // RB) - 1)
    def _():
        pooled = psum[...] / jnp.maximum(cnt[...], 1.0)
        xf = (
            jnp.dot(pooled, l1w_ref[...], preferred_element_type=jnp.float32)
            + l1b_ref[...]
        )
        out_ref[...] = (
            jnp.dot(
                jnp.maximum(xf, 0.0),
                l2w_ref[...],
                preferred_element_type=jnp.float32,
            )
            + l2b_ref[...]
        )


def _call_e(parts, u2, dinv, b2, batch3, lin1_W, lin1_b, lin2_W, lin2_b):
    grid = (N // RB,)
    return pl.pallas_call(
        _e_body,
        grid=grid,
        in_specs=[
            pl.BlockSpec((NC, RB, HD), lambda i: (0, i, 0)),
            pl.BlockSpec((NC, RB, HD), lambda i: (0, i, 0)),
            pl.BlockSpec((RB, 1), lambda i: (i, 0)),
            pl.BlockSpec((1, H), lambda i: (0, 0)),
            pl.BlockSpec((1, 1, RB), lambda i: (i, 0, 0)),
            pl.BlockSpec((H, D), lambda i: (0, 0)),
            pl.BlockSpec((1, D), lambda i: (0, 0)),
            pl.BlockSpec((D, OUT), lambda i: (0, 0)),
            pl.BlockSpec((1, OUT), lambda i: (0, 0)),
        ],
        out_specs=pl.BlockSpec((G, OUT), lambda i: (0, 0)),
        out_shape=jax.ShapeDtypeStruct((G, OUT), jnp.float32),
        scratch_shapes=[
            pltpu.VMEM((G, H), jnp.float32),
            pltpu.VMEM((G, 1), jnp.float32),
        ],
    )(parts, u2, dinv, b2, batch3, lin1_W, lin1_b, lin2_W, lin2_b)


def kernel(x, edge_index, batch, W1, b1, W2, b2, lin1_W, lin1_b, lin2_W,
           lin2_b):
    src = edge_index[0].astype(jnp.int32)
    dst = edge_index[1].astype(jnp.int32)
    src_r = src.reshape(NS, NCHUNK, CHA)
    dst_r = dst.reshape(NS, NCHUNK, CHA)
    dst_a = jnp.concatenate(
        [dst.reshape(NW, EPW), jnp.full((NW, PAD_A), N, jnp.int32)], axis=1
    ).reshape(NW, NCHUNK_A, CH)

    zcol = jnp.zeros((N, DW), jnp.float32)
    onescol = jnp.ones((CH, DW), jnp.float32)
    zrows = jnp.zeros((ROWS_PER_TILE, HD), jnp.float32)

    pdeg = _deg_call(dst_a, zcol, onescol)
    u1, dinv = _call_b(pdeg, x, W1)
    p1 = _agg_call(u1, src_r, dst_r, zrows)
    u2 = _call_d(p1, u1, dinv, b1.reshape(1, H), W2)
    p2 = _agg_call(u2, src_r, dst_r, zrows)
    out = _call_e(
        p2, u2, dinv, b2.reshape(1, H),
        batch.reshape(N // RB, 1, RB).astype(jnp.int32),
        lin1_W, lin1_b.reshape(1, D), lin2_W, lin2_b.reshape(1, OUT),
    )
    return out


# speedup vs baseline: 1.9334x; 1.0063x over previous
import jax
import jax.numpy as jnp
from jax import lax
from jax.experimental import pallas as pl
from jax.experimental.pallas import tpu as pltpu
from jax.experimental.pallas import tpu_sc as plsc

N = 10000
E = 320000
D = 128
H = 128
OUT = 32
G = 64

try:
    _info = plsc.get_sparse_core_info()
    NC = _info.num_cores
    NS = _info.num_subcores
except ValueError:
    NC = 2
    NS = 16
NW = NC * NS

HD = D // NC
CH = 128
CHA = 100
EPT = E // NS
NCHUNK = 200
PAD_E = NCHUNK * CHA - EPT
NBUF = 5
ACC_ROWS = N + 8
ROWS_PER_TILE = 640
TAIL_ROWS = N - (NS - 1) * ROWS_PER_TILE

_mesh = plsc.VectorSubcoreMesh(
    core_axis_name="c", subcore_axis_name="s", num_cores=NC, num_subcores=NS
)


EPW = E // NW
NCHUNK_A = 79
PAD_A = NCHUNK_A * CH - EPW
DW = 16


def _deg_body(dst_r, zcol, onescol, pdeg, dstbuf, onesbuf, acc, sem0, sem1):
    c = lax.axis_index("c")
    s = lax.axis_index("s")
    w = c * NS + s

    @pl.when(s == 0)
    def _():
        pltpu.sync_copy(zcol, acc.at[pl.ds(0, N)])

    pltpu.sync_copy(dst_r.at[w], dstbuf)
    pltpu.sync_copy(onescol, onesbuf)
    plsc.subcore_barrier()

    sems = [sem0, sem1]

    def fire(g, b):
        pltpu.async_copy(onesbuf, acc.at[dstbuf.at[g]], sems[b], add=True)

    def drain(b):
        pltpu.make_async_copy(onesbuf, acc.at[dstbuf.at[0]], sems[b]).wait()

    fire(0, 0)
    fire(1, 1)

    def loop(gi, carry):
        for b in range(2):
            g = 2 + gi * 2 + b
            drain(b)
            fire(g, b)
        return carry

    lax.fori_loop(0, (NCHUNK_A - 2) // 2, loop, 0)
    for r in range((NCHUNK_A - 2) % 2):
        drain(r)
        fire(NCHUNK_A - 1, r)
    drain(0)
    drain(1)

    plsc.subcore_barrier()

    @pl.when(s == 0)
    def _():
        pltpu.sync_copy(acc.at[pl.ds(0, N)], pdeg.at[c])


_deg_call = pl.kernel(
    _deg_body,
    out_type=jax.ShapeDtypeStruct((NC, N, DW), jnp.float32),
    mesh=_mesh,
    scratch_types=[
        pltpu.VMEM((NCHUNK_A, CH), jnp.int32),
        pltpu.VMEM((CH, DW), jnp.float32),
        pltpu.VMEM_SHARED((ACC_ROWS, DW), jnp.float32),
        pltpu.SemaphoreType.DMA,
        pltpu.SemaphoreType.DMA,
    ],
    compiler_params=pltpu.CompilerParams(use_tc_tiling_on_sc=False),
)


def _agg_body(uh, src_r, dst_r, zrows, parts, srcbuf, dstbuf, rows, acc,
              g0, g1, g2, g3, g4):
    c = lax.axis_index("c")
    s = lax.axis_index("s")

    tab = uh.at[c]

    @pl.when(s < NS - 1)
    def _():
        pltpu.sync_copy(zrows,
                        acc.at[pl.ds(s * ROWS_PER_TILE, ROWS_PER_TILE)])

    @pl.when(s == NS - 1)
    def _():
        pltpu.sync_copy(
            zrows.at[pl.ds(0, TAIL_ROWS)],
            acc.at[pl.ds((NS - 1) * ROWS_PER_TILE, TAIL_ROWS)],
        )

    pltpu.sync_copy(src_r.at[s], srcbuf)
    pltpu.sync_copy(dst_r.at[s], dstbuf)
    plsc.subcore_barrier()

    gsems = [g0, g1, g2, g3, g4]

    def fire(g, b):
        pltpu.async_copy(tab.at[srcbuf.at[g]], rows.at[b], gsems[b])

    def wait_gather(b):
        pltpu.make_async_copy(tab.at[pl.ds(0, CHA)], rows.at[b],
                              gsems[b]).wait()

    def sync_scatter(g, b):
        pltpu.sync_copy(rows.at[b], acc.at[dstbuf.at[g]], add=True)

    for b in range(NBUF):
        fire(b, b)

    def loop(gi, carry):
        for b in range(NBUF):
            g = gi * NBUF + b
            wait_gather(b)
            sync_scatter(g, b)
            fire(g + NBUF, b)
        return carry

    lax.fori_loop(0, NCHUNK // NBUF - 1, loop, 0)
    for b in range(NBUF):
        g = NCHUNK - NBUF + b
        wait_gather(b)
        sync_scatter(g, b)

    plsc.subcore_barrier()
    tail = N - (NS - 1) * ROWS_PER_TILE

    @pl.when(s < NS - 1)
    def _():
        pltpu.sync_copy(
            acc.at[pl.ds(s * ROWS_PER_TILE, ROWS_PER_TILE)],
            parts.at[c, pl.ds(s * ROWS_PER_TILE, ROWS_PER_TILE), :],
        )

    @pl.when(s == NS - 1)
    def _():
        pltpu.sync_copy(
            acc.at[pl.ds((NS - 1) * ROWS_PER_TILE, tail)],
            parts.at[c, pl.ds((NS - 1) * ROWS_PER_TILE, tail), :],
        )


_agg_call = pl.kernel(
    _agg_body,
    out_type=jax.ShapeDtypeStruct((NC, N, HD), jnp.float32),
    mesh=_mesh,
    scratch_types=[
        pltpu.VMEM((NCHUNK, CHA), jnp.int32),
        pltpu.VMEM((NCHUNK, CHA), jnp.int32),
        pltpu.VMEM((NBUF, CHA, HD), jnp.float32),
        pltpu.VMEM_SHARED((ACC_ROWS, HD), jnp.float32),
    ] + [pltpu.SemaphoreType.DMA] * 5,
    compiler_params=pltpu.CompilerParams(use_tc_tiling_on_sc=False),
)


RB = 2000


def _b_body(pd_ref, x_ref, w_ref, u_ref, dinv_ref):
    deg = pd_ref[0][:, 0:1] + pd_ref[1][:, 0:1] + 1.0
    dinv = lax.rsqrt(deg)
    res = (
        jnp.dot(x_ref[...], w_ref[...], preferred_element_type=jnp.float32)
        * dinv
    )
    u_ref[0] = res[:, :HD]
    u_ref[1] = res[:, HD:]
    dinv_ref[...] = dinv


def _call_b(pdeg, x, W1):
    grid = (N // RB,)
    return pl.pallas_call(
        _b_body,
        grid=grid,
        in_specs=[
            pl.BlockSpec((NC, RB, DW), lambda i: (0, i, 0)),
            pl.BlockSpec((RB, D), lambda i: (i, 0)),
            pl.BlockSpec((D, H), lambda i: (0, 0)),
        ],
        out_specs=[
            pl.BlockSpec((NC, RB, HD), lambda i: (0, i, 0)),
            pl.BlockSpec((RB, 1), lambda i: (i, 0)),
        ],
        out_shape=[
            jax.ShapeDtypeStruct((NC, N, HD), jnp.float32),
            jax.ShapeDtypeStruct((N, 1), jnp.float32),
        ],
    )(pdeg, x, W1)


def _d_body(p_ref, u_ref, dinv_ref, b1_ref, w2_ref, u2_ref):
    pres = jnp.concatenate(
        [p_ref[0] + u_ref[0], p_ref[1] + u_ref[1]], axis=1
    )
    h1 = jnp.maximum(dinv_ref[...] * pres + b1_ref[...], 0.0)
    res = (
        jnp.dot(h1, w2_ref[...], preferred_element_type=jnp.float32)
        * dinv_ref[...]
    )
    u2_ref[0] = res[:, :HD]
    u2_ref[1] = res[:, HD:]


def _call_d(parts, u1, dinv, b1, W2):
    grid = (N // RB,)
    return pl.pallas_call(
        _d_body,
        grid=grid,
        in_specs=[
            pl.BlockSpec((NC, RB, HD), lambda i: (0, i, 0)),
            pl.BlockSpec((NC, RB, HD), lambda i: (0, i, 0)),
            pl.BlockSpec((RB, 1), lambda i: (i, 0)),
            pl.BlockSpec((1, H), lambda i: (0, 0)),
            pl.BlockSpec((H, H), lambda i: (0, 0)),
        ],
        out_specs=pl.BlockSpec((NC, RB, HD), lambda i: (0, i, 0)),
        out_shape=jax.ShapeDtypeStruct((NC, N, HD), jnp.float32),
    )(parts, u1, dinv, b1, W2)


def _e_body(p_ref, u2_ref, dinv_ref, b2_ref, batch_ref, l1w_ref, l1b_ref,
            l2w_ref, l2b_ref, out_ref, psum, cnt):
    i = pl.program_id(0)
    pres = jnp.concatenate(
        [p_ref[0] + u2_ref[0], p_ref[1] + u2_ref[1]], axis=1
    )
    h2 = jnp.maximum(dinv_ref[...] * pres + b2_ref[...], 0.0)
    bt = batch_ref[0]
    oh = (lax.broadcasted_iota(jnp.int32, (G, RB), 0) == bt).astype(
        jnp.float32
    )
    ps = jnp.dot(oh, h2, preferred_element_type=jnp.float32)
    cn = jnp.sum(oh, axis=1, keepdims=True)

    @pl.when(i == 0)
    def _():
        psum[...] = ps
        cnt[...] = cn

    @pl.when(i > 0)
    def _():
        psum[...] += ps
        cnt[...] += cn

    @pl.when(i == (N // RB) - 1)
    def _():
        pooled = psum[...] / jnp.maximum(cnt[...], 1.0)
        xf = (
            jnp.dot(pooled, l1w_ref[...], preferred_element_type=jnp.float32)
            + l1b_ref[...]
        )
        out_ref[...] = (
            jnp.dot(
                jnp.maximum(xf, 0.0),
                l2w_ref[...],
                preferred_element_type=jnp.float32,
            )
            + l2b_ref[...]
        )


def _call_e(parts, u2, dinv, b2, batch3, lin1_W, lin1_b, lin2_W, lin2_b):
    grid = (N // RB,)
    return pl.pallas_call(
        _e_body,
        grid=grid,
        in_specs=[
            pl.BlockSpec((NC, RB, HD), lambda i: (0, i, 0)),
            pl.BlockSpec((NC, RB, HD), lambda i: (0, i, 0)),
            pl.BlockSpec((RB, 1), lambda i: (i, 0)),
            pl.BlockSpec((1, H), lambda i: (0, 0)),
            pl.BlockSpec((1, 1, RB), lambda i: (i, 0, 0)),
            pl.BlockSpec((H, D), lambda i: (0, 0)),
            pl.BlockSpec((1, D), lambda i: (0, 0)),
            pl.BlockSpec((D, OUT), lambda i: (0, 0)),
            pl.BlockSpec((1, OUT), lambda i: (0, 0)),
        ],
        out_specs=pl.BlockSpec((G, OUT), lambda i: (0, 0)),
        out_shape=jax.ShapeDtypeStruct((G, OUT), jnp.float32),
        scratch_shapes=[
            pltpu.VMEM((G, H), jnp.float32),
            pltpu.VMEM((G, 1), jnp.float32),
        ],
    )(parts, u2, dinv, b2, batch3, lin1_W, lin1_b, lin2_W, lin2_b)


def kernel(x, edge_index, batch, W1, b1, W2, b2, lin1_W, lin1_b, lin2_W,
           lin2_b):
    src = edge_index[0].astype(jnp.int32)
    dst = edge_index[1].astype(jnp.int32)
    src_r = src.reshape(NS, NCHUNK, CHA)
    dst_r = dst.reshape(NS, NCHUNK, CHA)
    dst_a = jnp.concatenate(
        [dst.reshape(NW, EPW), jnp.full((NW, PAD_A), N, jnp.int32)], axis=1
    ).reshape(NW, NCHUNK_A, CH)

    zcol = jnp.zeros((N, DW), jnp.float32)
    onescol = jnp.ones((CH, DW), jnp.float32)
    zrows = jnp.zeros((ROWS_PER_TILE, HD), jnp.float32)

    pdeg = _deg_call(dst_a, zcol, onescol)
    u1, dinv = _call_b(pdeg, x, W1)
    p1 = _agg_call(u1, src_r, dst_r, zrows)
    u2 = _call_d(p1, u1, dinv, b1.reshape(1, H), W2)
    p2 = _agg_call(u2, src_r, dst_r, zrows)
    out = _call_e(
        p2, u2, dinv, b2.reshape(1, H),
        batch.reshape(N // RB, 1, RB).astype(jnp.int32),
        lin1_W, lin1_b.reshape(1, D), lin2_W, lin2_b.reshape(1, OUT),
    )
    return out
